# Initial kernel scaffold; baseline (speedup 1.0000x reference)
import jax
import jax.numpy as jnp
from jax.experimental import pallas as pl


def kernel(inputs, emb_table, W, b):
    raise NotImplementedError("write your pallas kernel here")


# speedup vs baseline: 5.3627x; 5.3627x over previous
import functools

import jax
import jax.numpy as jnp
from jax import lax
from jax.experimental import pallas as pl
from jax.experimental.pallas import tpu as pltpu
from jax.experimental.pallas import tpu_sc as plsc

_NC = 2
_NS = 16
_NW = _NC * _NS
_CH = 12800


def _dg(vec, idx):
    return vec.at[idx].get(mode="promise_in_bounds")


def _body(idx_hbm, par_hbm, out_hbm, par_v, idx_v, out_v):
    wid = lax.axis_index("s") * _NC + lax.axis_index("c")
    n_per_w = idx_hbm.shape[0] // _NW
    nblk = n_per_w // _CH

    pltpu.sync_copy(par_hbm, par_v)
    lane = lax.iota(jnp.int32, 16)
    emb_v = par_v[pl.ds(0, 16)]
    w_v = par_v[pl.ds(16, 16)]
    b_v = par_v[pl.ds(32, 16)]
    k2 = (lane >> 2) * 2
    col = lane & 3
    lut = (_dg(emb_v, k2) * _dg(w_v, col)
           + _dg(emb_v, k2 + 1) * _dg(w_v, col + 4)
           + _dg(b_v, col))
    pats = [(lane >> 2) + 4 * g for g in range(4)]

    def blk_body(t, carry):
        base = wid * n_per_w + t * _CH
        pltpu.sync_copy(idx_hbm.at[pl.ds(base, _CH)], idx_v)

        def vec_body(i, c2):
            v4 = idx_v[pl.ds(i * 16, 16)] * 4
            for g in range(4):
                ag = _dg(v4, pats[g]) + col
                out_v[pl.ds(i * 64 + g * 16, 16)] = _dg(lut, ag)
            return c2

        lax.fori_loop(0, _CH // 16, vec_body, 0)
        pltpu.sync_copy(out_v, out_hbm.at[pl.ds(base * 4, _CH * 4)])
        return carry

    lax.fori_loop(0, nblk, blk_body, 0)


def kernel(inputs, emb_table, W, b):
    B, L = inputs.shape
    N = B * L
    idx_flat = inputs.reshape(N).astype(jnp.int32)
    par = jnp.zeros((48,), jnp.float32)
    par = par.at[0:6].set(emb_table.reshape(-1))
    par = par.at[16:24].set(W.reshape(-1))
    par = par.at[32:36].set(b)

    mesh = plsc.VectorSubcoreMesh(core_axis_name="c", subcore_axis_name="s")
    run = functools.partial(
        pl.kernel,
        mesh=mesh,
        out_type=jax.ShapeDtypeStruct((N * 4,), jnp.float32),
        scratch_types=[
            pltpu.VMEM((48,), jnp.float32),
            pltpu.VMEM((_CH,), jnp.int32),
            pltpu.VMEM((_CH * 4,), jnp.float32),
        ],
    )(_body)
    out = run(idx_flat, par)
    return out.reshape(B, L, 4)


# speedup vs baseline: 5.4150x; 1.0098x over previous
import functools

import jax
import jax.numpy as jnp
from jax import lax
from jax.experimental import pallas as pl
from jax.experimental.pallas import tpu as pltpu
from jax.experimental.pallas import tpu_sc as plsc

_NC = 2
_NS = 16
_NW = _NC * _NS
_CH = 6400
_UN = 8


def _dg(vec, idx):
    return vec.at[idx].get(mode="promise_in_bounds")


def _body(idx_hbm, par_hbm, out_hbm, par_v,
          idx_v0, idx_v1, out_v0, out_v1, s_i0, s_i1, s_o0, s_o1):
    wid = lax.axis_index("s") * _NC + lax.axis_index("c")
    n_per_w = idx_hbm.shape[0] // _NW
    nblk = n_per_w // _CH

    pltpu.sync_copy(par_hbm, par_v)
    lane = lax.iota(jnp.int32, 16)
    emb_v = par_v[pl.ds(0, 16)]
    w_v = par_v[pl.ds(16, 16)]
    b_v = par_v[pl.ds(32, 16)]
    k2 = (lane >> 2) * 2
    col = lane & 3
    lut = (_dg(emb_v, k2) * _dg(w_v, col)
           + _dg(emb_v, k2 + 1) * _dg(w_v, col + 4)
           + _dg(b_v, col))
    pats = [(lane >> 2) + 4 * g for g in range(4)]

    idx_bufs = [idx_v0, idx_v1]
    out_bufs = [out_v0, out_v1]
    si = [s_i0, s_i1]
    so = [s_o0, s_o1]
    base0 = wid * n_per_w

    icopy = [None, None]
    ocopy = [None, None]
    icopy[0] = pltpu.async_copy(idx_hbm.at[pl.ds(base0, _CH)], idx_bufs[0],
                                si[0])
    for t in range(nblk):
        cur = t & 1
        icopy[cur].wait()
        if t + 1 < nblk:
            icopy[1 - cur] = pltpu.async_copy(
                idx_hbm.at[pl.ds(base0 + (t + 1) * _CH, _CH)],
                idx_bufs[1 - cur], si[1 - cur])
        if t >= 2:
            ocopy[cur].wait()
        iv = idx_bufs[cur]
        ov = out_bufs[cur]

        def vec_body(i, c2, iv=iv, ov=ov):
            for u in range(_UN):
                j = i * _UN + u
                v4 = iv[pl.ds(j * 16, 16)] * 4
                for g in range(4):
                    ag = _dg(v4, pats[g]) + col
                    ov[pl.ds(j * 64 + g * 16, 16)] = _dg(lut, ag)
            return c2

        lax.fori_loop(0, _CH // (16 * _UN), vec_body, 0)
        ocopy[cur] = pltpu.async_copy(
            ov, out_hbm.at[pl.ds((base0 + t * _CH) * 4, _CH * 4)], so[cur])
    ocopy[0].wait()
    ocopy[1].wait()


def kernel(inputs, emb_table, W, b):
    B, L = inputs.shape
    N = B * L
    idx_flat = inputs.reshape(N).astype(jnp.int32)
    par = jnp.zeros((48,), jnp.float32)
    par = par.at[0:6].set(emb_table.reshape(-1))
    par = par.at[16:24].set(W.reshape(-1))
    par = par.at[32:36].set(b)

    mesh = plsc.VectorSubcoreMesh(core_axis_name="c", subcore_axis_name="s")
    run = functools.partial(
        pl.kernel,
        mesh=mesh,
        out_type=jax.ShapeDtypeStruct((N * 4,), jnp.float32),
        scratch_types=[
            pltpu.VMEM((48,), jnp.float32),
            pltpu.VMEM((_CH,), jnp.int32),
            pltpu.VMEM((_CH,), jnp.int32),
            pltpu.VMEM((_CH * 4,), jnp.float32),
            pltpu.VMEM((_CH * 4,), jnp.float32),
            pltpu.SemaphoreType.DMA,
            pltpu.SemaphoreType.DMA,
            pltpu.SemaphoreType.DMA,
            pltpu.SemaphoreType.DMA,
        ],
    )(_body)
    out = run(idx_flat, par)
    return out.reshape(B, L, 4)


# speedup vs baseline: 56.0215x; 10.3456x over previous
import functools

import jax
import jax.numpy as jnp
from jax import lax
from jax.experimental import pallas as pl
from jax.experimental.pallas import tpu as pltpu
from jax.experimental.pallas import tpu_sc as plsc

_NC = 2
_NS = 16
_NW = _NC * _NS
_L = 200
_RB = 32
_NVEC = 13


def _dg(vec, idx):
    return vec.at[idx].get(mode="promise_in_bounds")


def _body(idx_hbm, par_hbm, out_hbm, par_v,
          idx_v0, idx_v1, out_v0, out_v1, s_i0, s_i1, s_o0, s_o1):
    wid = lax.axis_index("s") * _NC + lax.axis_index("c")
    rows_per_w = idx_hbm.shape[0] // (_L * _NW)
    nblk = rows_per_w // _RB

    pltpu.sync_copy(par_hbm, par_v)
    lane = lax.iota(jnp.int32, 16)
    emb_v = par_v[pl.ds(0, 16)]
    w_v = par_v[pl.ds(16, 16)]
    b_v = par_v[pl.ds(32, 16)]
    k2 = jnp.minimum(lane, 7) * 2
    luts = []
    for c in range(4):
        cc = jnp.full((16,), c, jnp.int32)
        luts.append(_dg(emb_v, k2) * _dg(w_v, cc)
                    + _dg(emb_v, k2 + 1) * _dg(w_v, cc + 4)
                    + _dg(b_v, cc))

    idx_bufs = [idx_v0, idx_v1]
    out_bufs = [out_v0, out_v1]
    si = [s_i0, s_i1]
    so = [s_o0, s_o1]
    ibase = wid * rows_per_w * _L
    obase = wid * rows_per_w * _L * 4

    icopy = [None, None]
    ocopy = [None, None]
    icopy[0] = pltpu.async_copy(idx_hbm.at[pl.ds(ibase, _RB * _L)],
                                idx_bufs[0].at[pl.ds(0, _RB * _L)], si[0])
    for t in range(nblk):
        cur = t & 1
        icopy[cur].wait()
        if t + 1 < nblk:
            icopy[1 - cur] = pltpu.async_copy(
                idx_hbm.at[pl.ds(ibase + (t + 1) * _RB * _L, _RB * _L)],
                idx_bufs[1 - cur].at[pl.ds(0, _RB * _L)], si[1 - cur])
        if t >= 2:
            ocopy[cur].wait()
        iv = idx_bufs[cur]
        ov = out_bufs[cur]

        def row_body(r, carry, iv=iv, ov=ov):
            for c in range(4):
                for j in range(_NVEC):
                    v = iv[pl.ds(r * _L + j * 16, 16)]
                    ov[pl.ds(r * _L * 4 + c * _L + j * 16, 16)] = \
                        _dg(luts[c], v)
            return carry

        lax.fori_loop(0, _RB, row_body, 0)
        ocopy[cur] = pltpu.async_copy(
            ov.at[pl.ds(0, _RB * _L * 4)],
            out_hbm.at[pl.ds(obase + t * _RB * _L * 4, _RB * _L * 4)],
            so[cur])
    ocopy[0].wait()
    ocopy[1].wait()


def kernel(inputs, emb_table, W, b):
    B, L = inputs.shape
    N = B * L
    idx_flat = inputs.reshape(N).astype(jnp.int32)
    par = jnp.zeros((48,), jnp.float32)
    par = par.at[0:6].set(emb_table.reshape(-1))
    par = par.at[16:24].set(W.reshape(-1))
    par = par.at[32:36].set(b)

    mesh = plsc.VectorSubcoreMesh(core_axis_name="c", subcore_axis_name="s")
    run = functools.partial(
        pl.kernel,
        mesh=mesh,
        out_type=jax.ShapeDtypeStruct((N * 4,), jnp.float32),
        scratch_types=[
            pltpu.VMEM((48,), jnp.float32),
            pltpu.VMEM((_RB * _L + 8,), jnp.int32),
            pltpu.VMEM((_RB * _L + 8,), jnp.int32),
            pltpu.VMEM((_RB * _L * 4 + 8,), jnp.float32),
            pltpu.VMEM((_RB * _L * 4 + 8,), jnp.float32),
            pltpu.SemaphoreType.DMA,
            pltpu.SemaphoreType.DMA,
            pltpu.SemaphoreType.DMA,
            pltpu.SemaphoreType.DMA,
        ],
    )(_body)
    out = run(idx_flat, par)
    return jnp.swapaxes(out.reshape(B, 4, L), 1, 2)
